# baseline (speedup 1.0000x reference)
import functools

import jax
import jax.numpy as jnp
from jax import lax
from jax.experimental import pallas as pl
from jax.experimental.pallas import tpu as pltpu
from jax.experimental.pallas import tpu_sc as plsc

NC = 2
NS = 16
LANES = 16
NW = NC * NS

B = 1024
SEQ = 200
D = 128

ROWS_PER_W = B // NW
FLAT_PER_W = ROWS_PER_W * SEQ
GCHUNK = 128
N_GCHUNKS = FLAT_PER_W // GCHUNK
N_VCHUNKS = FLAT_PER_W // LANES
IDX_ROWS = FLAT_PER_W // GCHUNK


def _vgather16(vec, idx):
    dnums = lax.GatherDimensionNumbers(
        offset_dims=(), collapsed_slice_dims=(0,), start_index_map=(0,)
    )
    return lax.gather(
        vec, idx[:, None], dnums, slice_sizes=(1,),
        mode=lax.GatherScatterMode.PROMISE_IN_BOUNDS,
    )


def _sc_body(len_hbm, pe_hbm, out_hbm, pos_hbm, len_v, idx_v, rows_v, sem):
    wid = lax.axis_index("s") * NC + lax.axis_index("c")
    flat_base = wid * FLAT_PER_W
    row_base = wid * ROWS_PER_W

    pltpu.sync_copy(len_hbm.at[pl.ds(row_base, ROWS_PER_W)], len_v)

    iota = lax.iota(jnp.int32, LANES)

    def vchunk(k, carry):
        p = flat_base + k * LANES + iota
        b_loc = lax.div(p, SEQ) - row_base
        j1 = lax.rem(p, SEQ) + 1
        lo16 = len_v[pl.ds(0, LANES)]
        hi16 = len_v[pl.ds(LANES, LANES)]
        lens = jnp.where(
            b_loc < LANES,
            _vgather16(lo16, jnp.minimum(b_loc, LANES - 1)),
            _vgather16(hi16, jnp.maximum(b_loc - LANES, 0)),
        )
        posv = jnp.where(j1 <= lens, j1, 0)
        idx_v[k // 8, pl.ds((k % 8) * LANES, LANES)] = posv
        return carry

    lax.fori_loop(0, N_VCHUNKS, vchunk, 0)

    pltpu.sync_copy(idx_v, pos_hbm.at[wid])

    def gchunk(c, carry):
        pltpu.async_copy(pe_hbm.at[idx_v.at[c]], rows_v, sem).wait()
        pltpu.sync_copy(rows_v, out_hbm.at[pl.ds(flat_base + c * GCHUNK, GCHUNK)])
        return carry

    lax.fori_loop(0, N_GCHUNKS, gchunk, 0)


@functools.partial(jax.jit, static_argnames=())
def _run(lens, pe):
    mesh = plsc.VectorSubcoreMesh(
        core_axis_name="c", subcore_axis_name="s", num_cores=NC, num_subcores=NS
    )
    out_flat, pos_flat = pl.kernel(
        _sc_body,
        out_type=[
            jax.ShapeDtypeStruct((B * SEQ, D), jnp.float32),
            jax.ShapeDtypeStruct((NW, IDX_ROWS, GCHUNK), jnp.int32),
        ],
        mesh=mesh,
        scratch_types=[
            pltpu.VMEM((ROWS_PER_W,), jnp.int32),
            pltpu.VMEM((IDX_ROWS, GCHUNK), jnp.int32),
            pltpu.VMEM((GCHUNK, D), jnp.float32),
            pltpu.SemaphoreType.DMA,
        ],
    )(lens, pe)
    return out_flat, pos_flat


def kernel(input_len, position_encoding):
    lens = input_len.astype(jnp.int32)
    out_flat, pos_flat = _run(lens, position_encoding)
    positions = out_flat.reshape(B, SEQ, D)
    input_pos = pos_flat.reshape(B, SEQ)
    return positions, input_pos
